# Initial kernel scaffold; baseline (speedup 1.0000x reference)
import jax
import jax.numpy as jnp
from jax.experimental import pallas as pl


def kernel(query, key, roi_mask):
    raise NotImplementedError("write your pallas kernel here")


# speedup vs baseline: 7.9763x; 7.9763x over previous
import jax
import jax.numpy as jnp
from jax import lax
from jax.experimental import pallas as pl

_H = 16
_DK = 64


def _cossim_body(q_ref, k_ref, o_ref):
    q = q_ref[:]
    k = k_ref[:]
    ch = q.shape[1]
    io_c = lax.broadcasted_iota(jnp.int32, (ch, _H), 0)
    io_h = lax.broadcasted_iota(jnp.int32, (ch, _H), 1)
    sel = (io_c // _DK == io_h).astype(jnp.float32)
    dot = jnp.dot(q * k, sel, precision=lax.Precision.HIGHEST)
    qq = jnp.dot(q * q, sel, precision=lax.Precision.HIGHEST)
    kk = jnp.dot(k * k, sel, precision=lax.Precision.HIGHEST)
    eps = 1e-8
    qn = jnp.maximum(jnp.sqrt(qq), eps)
    kn = jnp.maximum(jnp.sqrt(kk), eps)
    o_ref[:] = jnp.maximum(dot / (qn * kn), 0.0)


def _mask_body(a_ref, m_ref):
    v = a_ref[:]
    X = v.shape[1]
    rowio = lax.broadcasted_iota(jnp.int32, v.shape, 1)
    taken = jnp.zeros(v.shape, jnp.bool_)
    for _ in range(4):
        m = jnp.max(v, axis=1, keepdims=True)
        ismax = v == m
        jstar = jnp.min(jnp.where(ismax, rowio, X), axis=1, keepdims=True)
        pick = rowio == jstar
        taken = jnp.logical_or(taken, pick)
        v = jnp.where(pick, -1.0, v)
    tk = taken.astype(jnp.float32)
    mg = jnp.max(tk, axis=0)
    mh = jnp.max(mg, axis=1, keepdims=True)

    @pl.when(pl.program_id(0) == 0)
    def _init():
        m_ref[:] = jnp.zeros_like(m_ref)

    m_ref[:] = jnp.maximum(m_ref[:], mh)


def _apply_body(a_ref, r_ref, m_ref, o_ref):
    X = m_ref.shape[0]
    mask = m_ref[:].reshape(1, X, 1)
    o_ref[:] = a_ref[:] * r_ref[:] * mask


def kernel(query, key, roi_mask):
    B, num, X, ch = query.shape
    R = B * num * X
    qf = query.reshape(R, ch)
    kf = key.reshape(R, ch)

    BR = 1024
    attn = pl.pallas_call(
        _cossim_body,
        grid=(R // BR,),
        in_specs=[
            pl.BlockSpec((BR, ch), lambda i: (i, 0)),
            pl.BlockSpec((BR, ch), lambda i: (i, 0)),
        ],
        out_specs=pl.BlockSpec((BR, _H), lambda i: (i, 0)),
        out_shape=jax.ShapeDtypeStruct((R, _H), jnp.float32),
    )(qf, kf)

    attn3 = attn.reshape(B * num, X, _H)

    G = 32
    mask = pl.pallas_call(
        _mask_body,
        grid=(B * num // G,),
        in_specs=[pl.BlockSpec((G, X, _H), lambda i: (i, 0, 0))],
        out_specs=pl.BlockSpec((X, 1), lambda i: (0, 0)),
        out_shape=jax.ShapeDtypeStruct((X, 1), jnp.float32),
    )(attn3)

    G2 = 64
    rf = roi_mask.reshape(B * num, X, 1)
    out = pl.pallas_call(
        _apply_body,
        grid=(B * num // G2,),
        in_specs=[
            pl.BlockSpec((G2, X, _H), lambda i: (i, 0, 0)),
            pl.BlockSpec((G2, X, 1), lambda i: (i, 0, 0)),
            pl.BlockSpec((X, 1), lambda i: (0, 0)),
        ],
        out_specs=pl.BlockSpec((G2, X, _H), lambda i: (i, 0, 0)),
        out_shape=jax.ShapeDtypeStruct((B * num, X, _H), jnp.float32),
    )(attn3, rf, mask)

    return out.reshape(B, num, X, _H)


# speedup vs baseline: 17.4916x; 2.1930x over previous
import jax
import jax.numpy as jnp
from jax import lax
from jax.experimental import pallas as pl

_H = 16
_DK = 64


def _main_body(q_ref, k_ref, o_ref, m_ref):
    q = q_ref[:]
    k = k_ref[:]
    ch = q.shape[1]
    io_c = lax.broadcasted_iota(jnp.int32, (ch, _H), 0)
    io_h = lax.broadcasted_iota(jnp.int32, (ch, _H), 1)
    sel = (io_c // _DK == io_h).astype(jnp.bfloat16)

    def seg_sum(x):
        hi = x.astype(jnp.bfloat16)
        lo = (x - hi.astype(jnp.float32)).astype(jnp.bfloat16)
        return (jnp.dot(hi, sel, preferred_element_type=jnp.float32)
                + jnp.dot(lo, sel, preferred_element_type=jnp.float32))

    dot = seg_sum(q * k)
    qq = seg_sum(q * q)
    kk = seg_sum(k * k)
    eps = 1e-8
    qn = jnp.maximum(jnp.sqrt(qq), eps)
    kn = jnp.maximum(jnp.sqrt(kk), eps)
    attn = jnp.maximum(dot / (qn * kn), 0.0)
    o_ref[:] = attn

    ngrp = attn.shape[0] // 128
    at = jnp.swapaxes(attn.reshape(ngrp, 128, _H), 1, 2)
    rowio = lax.broadcasted_iota(jnp.int32, at.shape, 2)
    taken = jnp.zeros(at.shape, jnp.bool_)
    for _ in range(4):
        m = jnp.max(at, axis=2, keepdims=True)
        ismax = at == m
        jstar = jnp.min(jnp.where(ismax, rowio, 128), axis=2, keepdims=True)
        pick = rowio == jstar
        taken = jnp.logical_or(taken, pick)
        at = jnp.where(pick, -1.0, at)
    tk = taken.astype(jnp.float32)
    mh = jnp.max(jnp.max(tk, axis=0), axis=0).reshape(1, 128)

    @pl.when(pl.program_id(0) == 0)
    def _init():
        m_ref[:] = jnp.zeros_like(m_ref)

    m_ref[:] = jnp.maximum(m_ref[:], mh)


def _apply_body(a_ref, r_ref, m_ref, o_ref):
    X = m_ref.shape[1]
    mask = jnp.swapaxes(m_ref[:], 0, 1).reshape(1, X, 1)
    o_ref[:] = a_ref[:] * r_ref[:] * mask


def kernel(query, key, roi_mask):
    B, num, X, ch = query.shape
    R = B * num * X
    qf = query.reshape(R, ch)
    kf = key.reshape(R, ch)

    BR = 1024
    attn, mask = pl.pallas_call(
        _main_body,
        grid=(R // BR,),
        in_specs=[
            pl.BlockSpec((BR, ch), lambda i: (i, 0)),
            pl.BlockSpec((BR, ch), lambda i: (i, 0)),
        ],
        out_specs=[
            pl.BlockSpec((BR, _H), lambda i: (i, 0)),
            pl.BlockSpec((1, X), lambda i: (0, 0)),
        ],
        out_shape=[
            jax.ShapeDtypeStruct((R, _H), jnp.float32),
            jax.ShapeDtypeStruct((1, X), jnp.float32),
        ],
    )(qf, kf)

    attn3 = attn.reshape(B * num, X, _H)

    G2 = 64
    rf = roi_mask.reshape(B * num, X, 1)
    out = pl.pallas_call(
        _apply_body,
        grid=(B * num // G2,),
        in_specs=[
            pl.BlockSpec((G2, X, _H), lambda i: (i, 0, 0)),
            pl.BlockSpec((G2, X, 1), lambda i: (i, 0, 0)),
            pl.BlockSpec((1, X), lambda i: (0, 0)),
        ],
        out_specs=pl.BlockSpec((G2, X, _H), lambda i: (i, 0, 0)),
        out_shape=jax.ShapeDtypeStruct((B * num, X, _H), jnp.float32),
    )(attn3, rf, mask)

    return out.reshape(B, num, X, _H)


# speedup vs baseline: 21.3349x; 1.2197x over previous
import jax
import jax.numpy as jnp
from jax import lax
from jax.experimental import pallas as pl

_H = 16
_DK = 64


def _main_body(q_ref, k_ref, o_ref, m_ref):
    q = q_ref[:]
    k = k_ref[:]
    ch = q.shape[1]
    io_c = lax.broadcasted_iota(jnp.int32, (ch, _H), 0)
    io_h = lax.broadcasted_iota(jnp.int32, (ch, _H), 1)
    sel = (io_c // _DK == io_h).astype(jnp.bfloat16)

    def seg_sum(x):
        return jnp.dot(x.astype(jnp.bfloat16), sel,
                       preferred_element_type=jnp.float32)

    dot = seg_sum(q * k)
    qq = seg_sum(q * q)
    kk = seg_sum(k * k)
    eps = 1e-8
    qn = jnp.maximum(jnp.sqrt(qq), eps)
    kn = jnp.maximum(jnp.sqrt(kk), eps)
    attn = jnp.maximum(dot / (qn * kn), 0.0)
    o_ref[:] = attn

    ngrp = attn.shape[0] // 128
    at = jnp.swapaxes(attn.reshape(ngrp, 128, _H), 1, 2)
    rowio = lax.broadcasted_iota(jnp.int32, at.shape, 2)
    taken = jnp.zeros(at.shape, jnp.bool_)
    for _ in range(4):
        m = jnp.max(at, axis=2, keepdims=True)
        ismax = at == m
        jstar = jnp.min(jnp.where(ismax, rowio, 128), axis=2, keepdims=True)
        pick = rowio == jstar
        taken = jnp.logical_or(taken, pick)
        at = jnp.where(pick, -1.0, at)
    tk = taken.astype(jnp.float32)
    mh = jnp.max(jnp.max(tk, axis=0), axis=0).reshape(1, 128)

    @pl.when(pl.program_id(0) == 0)
    def _init():
        m_ref[:] = jnp.zeros_like(m_ref)

    m_ref[:] = jnp.maximum(m_ref[:], mh)


def _apply_body(a_ref, r_ref, m_ref, o_ref):
    X = m_ref.shape[1]
    mask = jnp.swapaxes(m_ref[:], 0, 1).reshape(1, X, 1)
    o_ref[:] = a_ref[:] * r_ref[:] * mask


def kernel(query, key, roi_mask):
    B, num, X, ch = query.shape
    R = B * num * X
    qf = query.reshape(R, ch)
    kf = key.reshape(R, ch)

    BR = 1024
    attn, mask = pl.pallas_call(
        _main_body,
        grid=(R // BR,),
        in_specs=[
            pl.BlockSpec((BR, ch), lambda i: (i, 0)),
            pl.BlockSpec((BR, ch), lambda i: (i, 0)),
        ],
        out_specs=[
            pl.BlockSpec((BR, _H), lambda i: (i, 0)),
            pl.BlockSpec((1, X), lambda i: (0, 0)),
        ],
        out_shape=[
            jax.ShapeDtypeStruct((R, _H), jnp.float32),
            jax.ShapeDtypeStruct((1, X), jnp.float32),
        ],
    )(qf, kf)

    attn3 = attn.reshape(B * num, X, _H)

    G2 = 64
    rf = roi_mask.reshape(B * num, X, 1)
    out = pl.pallas_call(
        _apply_body,
        grid=(B * num // G2,),
        in_specs=[
            pl.BlockSpec((G2, X, _H), lambda i: (i, 0, 0)),
            pl.BlockSpec((G2, X, 1), lambda i: (i, 0, 0)),
            pl.BlockSpec((1, X), lambda i: (0, 0)),
        ],
        out_specs=pl.BlockSpec((G2, X, _H), lambda i: (i, 0, 0)),
        out_shape=jax.ShapeDtypeStruct((B * num, X, _H), jnp.float32),
    )(attn3, rf, mask)

    return out.reshape(B, num, X, _H)


# speedup vs baseline: 25.8263x; 1.2105x over previous
import jax
import jax.numpy as jnp
from jax import lax
from jax.experimental import pallas as pl

_H = 16
_DK = 64


def _main_body(q_ref, k_ref, o_ref, m_ref):
    q = q_ref[:]
    k = k_ref[:]
    ch = q.shape[1]
    io_c = lax.broadcasted_iota(jnp.int32, (ch, _H), 0)
    io_h = lax.broadcasted_iota(jnp.int32, (ch, _H), 1)
    sel = (io_c // _DK == io_h).astype(jnp.bfloat16)

    qb = q.astype(jnp.bfloat16)
    kb = k.astype(jnp.bfloat16)

    def seg_sum(x):
        return jnp.dot(x, sel, preferred_element_type=jnp.float32)

    dot = seg_sum(qb * kb)
    qq = seg_sum(qb * qb)
    kk = seg_sum(kb * kb)
    eps = 1e-8
    qn = jnp.maximum(jnp.sqrt(qq), eps)
    kn = jnp.maximum(jnp.sqrt(kk), eps)
    attn = jnp.maximum(dot / (qn * kn), 0.0)

    ngrp = attn.shape[0] // 128
    at = jnp.swapaxes(attn.reshape(ngrp, 128, _H), 1, 2)
    o_ref[:] = at.reshape(ngrp * _H, 128)

    rowio = lax.broadcasted_iota(jnp.int32, at.shape, 2)
    taken = jnp.zeros(at.shape, jnp.bool_)
    for _ in range(4):
        m = jnp.max(at, axis=2, keepdims=True)
        ismax = at == m
        jstar = jnp.min(jnp.where(ismax, rowio, 128), axis=2, keepdims=True)
        pick = rowio == jstar
        taken = jnp.logical_or(taken, pick)
        at = jnp.where(pick, -1.0, at)
    tk = taken.astype(jnp.float32)
    mh = jnp.max(jnp.max(tk, axis=0), axis=0).reshape(1, 128)

    @pl.when(pl.program_id(0) == 0)
    def _init():
        m_ref[:] = jnp.zeros_like(m_ref)

    m_ref[:] = jnp.maximum(m_ref[:], mh)


def _apply_body(a_ref, r_ref, m_ref, o_ref):
    nb = r_ref.shape[0]
    a = a_ref[:].reshape(nb, _H, 128)
    roi = jnp.broadcast_to(r_ref[:].reshape(nb, 1, 128), (nb, _H, 128))
    mask = m_ref[:].reshape(1, 1, 128)
    masked = a * roi * mask
    o_ref[:] = jnp.swapaxes(masked, 1, 2)


def kernel(query, key, roi_mask):
    B, num, X, ch = query.shape
    R = B * num * X
    BI = B * num
    qf = query.reshape(R, ch)
    kf = key.reshape(R, ch)

    BR = 1024
    attn_t, mask = pl.pallas_call(
        _main_body,
        grid=(R // BR,),
        in_specs=[
            pl.BlockSpec((BR, ch), lambda i: (i, 0)),
            pl.BlockSpec((BR, ch), lambda i: (i, 0)),
        ],
        out_specs=[
            pl.BlockSpec((BR // 128 * _H, 128), lambda i: (i, 0)),
            pl.BlockSpec((1, X), lambda i: (0, 0)),
        ],
        out_shape=[
            jax.ShapeDtypeStruct((BI * _H, 128), jnp.float32),
            jax.ShapeDtypeStruct((1, X), jnp.float32),
        ],
    )(qf, kf)

    NB = 128
    rf = roi_mask.reshape(BI, X)
    out = pl.pallas_call(
        _apply_body,
        grid=(BI // NB,),
        in_specs=[
            pl.BlockSpec((NB * _H, 128), lambda i: (i, 0)),
            pl.BlockSpec((NB, X), lambda i: (i, 0)),
            pl.BlockSpec((1, X), lambda i: (0, 0)),
        ],
        out_specs=pl.BlockSpec((NB, X, _H), lambda i: (i, 0, 0)),
        out_shape=jax.ShapeDtypeStruct((BI, X, _H), jnp.float32),
    )(attn_t, rf, mask)

    return out.reshape(B, num, X, _H)


# speedup vs baseline: 28.5150x; 1.1041x over previous
import jax
import jax.numpy as jnp
from jax import lax
from jax.experimental import pallas as pl

_H = 16
_DK = 64


def _main_body(q_ref, k_ref, o_ref, m_ref):
    q = q_ref[:]
    k = k_ref[:]
    ch = q.shape[1]
    io_c = lax.broadcasted_iota(jnp.int32, (ch, _H), 0)
    io_h = lax.broadcasted_iota(jnp.int32, (ch, _H), 1)
    sel = (io_c // _DK == io_h).astype(jnp.bfloat16)

    qb = q.astype(jnp.bfloat16)
    kb = k.astype(jnp.bfloat16)

    def seg_sum(x):
        return jnp.dot(x, sel, preferred_element_type=jnp.float32)

    dot = seg_sum(qb * kb)
    qq = seg_sum(qb * qb)
    kk = seg_sum(kb * kb)
    eps = 1e-8
    qn = jnp.maximum(jnp.sqrt(qq), eps)
    kn = jnp.maximum(jnp.sqrt(kk), eps)
    attn = jnp.maximum(dot / (qn * kn), 0.0)

    ngrp = attn.shape[0] // 128
    at = jnp.swapaxes(attn.reshape(ngrp, 128, _H), 1, 2)
    o_ref[:] = at.reshape(ngrp * _H, 128)

    rowio = lax.broadcasted_iota(jnp.int32, at.shape, 2)
    taken = jnp.zeros(at.shape, jnp.bool_)
    for _ in range(4):
        m = jnp.max(at, axis=2, keepdims=True)
        ismax = at == m
        jstar = jnp.min(jnp.where(ismax, rowio, 128), axis=2, keepdims=True)
        pick = rowio == jstar
        taken = jnp.logical_or(taken, pick)
        at = jnp.where(pick, -1.0, at)
    tk = taken.astype(jnp.float32)
    mh = jnp.max(jnp.max(tk, axis=0), axis=0).reshape(1, 128)

    @pl.when(pl.program_id(0) == 0)
    def _init():
        m_ref[:] = jnp.zeros_like(m_ref)

    m_ref[:] = jnp.maximum(m_ref[:], mh)


def _apply_body(a_ref, r_ref, m_ref, o_ref):
    nb = r_ref.shape[0]
    a = a_ref[:].reshape(nb, _H, 128)
    roi = jnp.broadcast_to(r_ref[:].reshape(nb, 1, 128), (nb, _H, 128))
    mask = m_ref[:].reshape(1, 1, 128)
    masked = a * roi * mask
    o_ref[:] = jnp.swapaxes(masked, 1, 2)


def kernel(query, key, roi_mask):
    B, num, X, ch = query.shape
    R = B * num * X
    BI = B * num
    qf = query.reshape(R, ch)
    kf = key.reshape(R, ch)

    BR = 2048
    attn_t, mask = pl.pallas_call(
        _main_body,
        grid=(R // BR,),
        in_specs=[
            pl.BlockSpec((BR, ch), lambda i: (i, 0)),
            pl.BlockSpec((BR, ch), lambda i: (i, 0)),
        ],
        out_specs=[
            pl.BlockSpec((BR // 128 * _H, 128), lambda i: (i, 0)),
            pl.BlockSpec((1, X), lambda i: (0, 0)),
        ],
        out_shape=[
            jax.ShapeDtypeStruct((BI * _H, 128), jnp.float32),
            jax.ShapeDtypeStruct((1, X), jnp.float32),
        ],
    )(qf, kf)

    NB = 128
    rf = roi_mask.reshape(BI, X)
    out = pl.pallas_call(
        _apply_body,
        grid=(BI // NB,),
        in_specs=[
            pl.BlockSpec((NB * _H, 128), lambda i: (i, 0)),
            pl.BlockSpec((NB, X), lambda i: (i, 0)),
            pl.BlockSpec((1, X), lambda i: (0, 0)),
        ],
        out_specs=pl.BlockSpec((NB, X, _H), lambda i: (i, 0, 0)),
        out_shape=jax.ShapeDtypeStruct((BI, X, _H), jnp.float32),
    )(attn_t, rf, mask)

    return out.reshape(B, num, X, _H)
